# baseline (speedup 1.0000x reference)
import functools

import jax
import jax.numpy as jnp
from jax import lax
from jax.experimental import pallas as pl
from jax.experimental.pallas import tpu as pltpu
from jax.experimental.pallas import tpu_sc as plsc

_N = 100000
_NSEG = 16
_C1 = 6400
_C2 = 3200
_L = 16

_mesh = plsc.VectorSubcoreMesh(core_axis_name="c", subcore_axis_name="s")


@functools.partial(
    pl.kernel,
    mesh=_mesh,
    compiler_params=pltpu.CompilerParams(needs_layout_passes=False),
    out_type=jax.ShapeDtypeStruct((_N,), jnp.float32),
    scratch_types=[
        pltpu.VMEM((_C1,), jnp.float32),
        pltpu.VMEM((_C2,), jnp.float32),
        pltpu.VMEM((2 * _L,), jnp.int32),
        pltpu.VMEM((_L,), jnp.float32),
        pltpu.VMEM((_L,), jnp.float32),
        pltpu.VMEM((16 * _L,), jnp.float32),
        pltpu.VMEM_SHARED((16 * _L,), jnp.float32),
        pltpu.SemaphoreType.DMA,
        pltpu.SemaphoreType.DMA,
        pltpu.SemaphoreType.DMA,
    ],
)
def _filtration_kernel(deg_hbm, pos_hbm, out_hbm,
                       chunk_v, out_v, pos_v, stage_v, inv_v, allp_v,
                       shared, sem, sema, semb):
    c = lax.axis_index("c")
    s = lax.axis_index("s")
    base1 = s * _C1
    last1 = s == (_NSEG - 1)

    pos_cp = pltpu.async_copy(pos_hbm.at[pl.ds(0, _L)],
                              pos_v.at[pl.ds(0, _L)], sem)
    cp_a = pltpu.async_copy(deg_hbm.at[pl.ds(base1, _C2)],
                            chunk_v.at[pl.ds(0, _C2)], sema)

    @pl.when(jnp.logical_not(last1))
    def _():
        pltpu.async_copy(deg_hbm.at[pl.ds(base1 + _C2, _C2)],
                         chunk_v.at[pl.ds(_C2, _C2)], semb)

    @pl.when(last1)
    def _():
        pltpu.async_copy(deg_hbm.at[pl.ds(_N - 800, 800)],
                         chunk_v.at[pl.ds(_C2, 800)], semb)

    pos_cp.wait()

    iota = lax.iota(jnp.int32, _L)
    ninf = jnp.full((_L,), -jnp.inf, dtype=jnp.float32)

    pos_vec = pos_v[pl.ds(0, _L)]
    pos_v[pl.ds(_L, _L)] = jnp.full((_L,), _N, dtype=jnp.int32)

    def seg_range(base, w):
        cnt_lo = plsc.all_reduce_population_count(pos_vec <= base)
        cnt_hi = plsc.all_reduce_population_count(pos_vec < base + w)
        return cnt_lo[0] - 1, cnt_hi[0] - 1

    def phase1_partials(wbase, wsize, boff):
        sf, sl = seg_range(wbase, wsize)

        def seg_body(seg, pvec):
            pp = plsc.load_gather(pos_v, [seg + jnp.minimum(iota, 1)])
            lo = jnp.clip(pp[0] - wbase, 0, wsize)
            hi = jnp.clip(pp[1] - wbase, lo, wsize)

            def seg_max(lo=lo, hi=hi):
                def masked_max(acc, j):
                    v = chunk_v[pl.ds(boff + j * _L, _L)]
                    idx = j * _L + iota
                    m = (idx >= lo) & (idx < hi)
                    return jnp.maximum(acc, jnp.where(m, v, ninf))

                acc = masked_max(ninf, lo // _L)
                acc = masked_max(acc, (hi - 1) // _L)
                a = (lo + _L - 1) // _L
                b = jnp.maximum(a, hi // _L)

                def body(j, acc):
                    return jnp.maximum(acc,
                                       chunk_v[pl.ds(boff + j * _L, _L)])

                acc = plsc.parallel_loop(a, b, 1, unroll=4,
                                         carry=acc)(body)
                return jnp.max(acc)

            segmax = lax.cond(lo < hi, seg_max, lambda: -jnp.inf)
            return jnp.where(iota == seg, segmax, pvec)

        return lax.fori_loop(sf, sl + 1, seg_body, ninf)

    cp_a.wait()
    pvec = phase1_partials(base1, _C2, 0)

    wb = jnp.where(last1, 800, _C2)

    @pl.when(jnp.logical_not(last1))
    def _():
        pltpu.make_async_copy(deg_hbm.at[pl.ds(base1 + _C2, _C2)],
                              chunk_v.at[pl.ds(_C2, _C2)], semb).wait()

    @pl.when(last1)
    def _():
        pltpu.make_async_copy(deg_hbm.at[pl.ds(_N - 800, 800)],
                              chunk_v.at[pl.ds(_C2, 800)], semb).wait()

    pvec = jnp.maximum(pvec, phase1_partials(base1 + _C2, wb, _C2))

    stage_v[...] = pvec
    pltpu.sync_copy(stage_v, shared.at[pl.ds(s * _L, _L)])
    plsc.subcore_barrier()
    pltpu.sync_copy(shared, allp_v)
    gmax = ninf
    for r in range(16):
        gmax = jnp.maximum(gmax, allp_v[pl.ds(r * _L, _L)])
    inv_v[...] = 1.0 / gmax

    off = c * _C2
    base2 = base1 + off
    last2 = last1 & (c == 1)
    w2 = jnp.where(last2, 800, _C2)
    sf2, sl2 = seg_range(base2, w2)

    def seg_body2(seg, carry):
        pp = plsc.load_gather(pos_v, [seg + jnp.minimum(iota, 1)])
        lo = jnp.clip(pp[0] - base2, 0, w2)
        hi = jnp.clip(pp[1] - base2, lo, w2)
        scale = plsc.load_gather(inv_v, [jnp.broadcast_to(seg, (_L,))])

        @pl.when(lo < hi)
        def _(lo=lo, hi=hi, scale=scale):
            def edge(j):
                v = chunk_v[pl.ds(off + j * _L, _L)]
                idx = j * _L + iota
                m = (idx >= lo) & (idx < hi)
                cur = out_v[pl.ds(j * _L, _L)]
                out_v[pl.ds(j * _L, _L)] = jnp.where(m, v * scale, cur)

            edge(lo // _L)
            edge((hi - 1) // _L)

            a = (lo + _L - 1) // _L
            b = jnp.maximum(a, hi // _L)

            def body2(j):
                out_v[pl.ds(j * _L, _L)] = (
                    chunk_v[pl.ds(off + j * _L, _L)] * scale)

            plsc.parallel_loop(a, b, 1, unroll=4)(body2)

        return carry

    lax.fori_loop(sf2, sl2 + 1, seg_body2, 0)

    @pl.when(jnp.logical_not(last2))
    def _():
        pltpu.sync_copy(out_v, out_hbm.at[pl.ds(base2, _C2)])

    @pl.when(last2)
    def _():
        pltpu.sync_copy(out_v.at[pl.ds(0, 800)],
                        out_hbm.at[pl.ds(_N - 800, 800)])


def kernel(node_deg, sample_pos):
    return _filtration_kernel(node_deg.astype(jnp.float32),
                              sample_pos.astype(jnp.int32))


# speedup vs baseline: 1.0261x; 1.0261x over previous
import functools

import jax
import jax.numpy as jnp
from jax import lax
from jax.experimental import pallas as pl
from jax.experimental.pallas import tpu as pltpu
from jax.experimental.pallas import tpu_sc as plsc

_N = 100000
_NSEG = 16
_C1 = 6400
_C2 = 3200
_L = 16

_mesh = plsc.VectorSubcoreMesh(core_axis_name="c", subcore_axis_name="s")


@functools.partial(
    pl.kernel,
    mesh=_mesh,
    compiler_params=pltpu.CompilerParams(needs_layout_passes=False),
    out_type=jax.ShapeDtypeStruct((_N,), jnp.float32),
    scratch_types=[
        pltpu.VMEM((_C1,), jnp.float32),
        pltpu.VMEM((_C2,), jnp.float32),
        pltpu.VMEM((2 * _L,), jnp.int32),
        pltpu.VMEM((_L,), jnp.float32),
        pltpu.VMEM((_L,), jnp.float32),
        pltpu.VMEM((16 * _L,), jnp.float32),
        pltpu.VMEM_SHARED((16 * _L,), jnp.float32),
        pltpu.SemaphoreType.DMA,
    ],
)
def _filtration_kernel(deg_hbm, pos_hbm, out_hbm,
                       chunk_v, out_v, pos_v, stage_v, inv_v, allp_v,
                       shared, sem):
    c = lax.axis_index("c")
    s = lax.axis_index("s")
    base1 = s * _C1
    last1 = s == (_NSEG - 1)

    pos_cp = pltpu.async_copy(pos_hbm.at[pl.ds(0, _L)],
                              pos_v.at[pl.ds(0, _L)], sem)

    @pl.when(jnp.logical_not(last1))
    def _():
        pltpu.sync_copy(deg_hbm.at[pl.ds(base1, _C1)], chunk_v)

    @pl.when(last1)
    def _():
        pltpu.sync_copy(deg_hbm.at[pl.ds(_N - 4000, 4000)],
                        chunk_v.at[pl.ds(0, 4000)])

    pos_cp.wait()

    iota = lax.iota(jnp.int32, _L)
    ninf = jnp.full((_L,), -jnp.inf, dtype=jnp.float32)

    pos_vec = pos_v[pl.ds(0, _L)]
    pos_v[pl.ds(_L, _L)] = jnp.full((_L,), _N, dtype=jnp.int32)

    def seg_range(base, w):
        cnt_lo = plsc.all_reduce_population_count(pos_vec <= base)
        cnt_hi = plsc.all_reduce_population_count(pos_vec < base + w)
        return cnt_lo[0] - 1, cnt_hi[0] - 1

    w1 = jnp.where(last1, 4000, _C1)

    sf1, sl1 = seg_range(base1, w1)

    def seg_body(seg, pvec):
        pp = plsc.load_gather(pos_v, [seg + jnp.minimum(iota, 1)])
        lo = jnp.clip(pp[0] - base1, 0, w1)
        hi = jnp.clip(pp[1] - base1, lo, w1)

        def seg_max(lo=lo, hi=hi):
            def masked_max(acc, j):
                v = chunk_v[pl.ds(j * _L, _L)]
                idx = j * _L + iota
                m = (idx >= lo) & (idx < hi)
                return jnp.maximum(acc, jnp.where(m, v, ninf))

            acc = masked_max(ninf, lo // _L)
            acc = masked_max(acc, (hi - 1) // _L)
            a = (lo + _L - 1) // _L
            b = jnp.maximum(a, hi // _L)

            def body(j, acc):
                return jnp.maximum(acc, chunk_v[pl.ds(j * _L, _L)])

            acc = plsc.parallel_loop(a, b, 1, unroll=8, carry=acc)(body)
            return jnp.max(acc)

        segmax = lax.cond(lo < hi, seg_max, lambda: -jnp.inf)
        return jnp.where(iota == seg, segmax, pvec)

    pvec = lax.fori_loop(sf1, sl1 + 1, seg_body, ninf)

    stage_v[...] = pvec
    pltpu.sync_copy(stage_v, shared.at[pl.ds(s * _L, _L)])
    plsc.subcore_barrier()
    pltpu.sync_copy(shared, allp_v)
    gmax = ninf
    for r in range(16):
        gmax = jnp.maximum(gmax, allp_v[pl.ds(r * _L, _L)])
    inv_v[...] = 1.0 / gmax

    off = c * _C2
    base2 = base1 + off
    last2 = last1 & (c == 1)
    w2 = jnp.where(last2, 800, _C2)
    sf2, sl2 = seg_range(base2, w2)

    def seg_body2(seg, carry):
        pp = plsc.load_gather(pos_v, [seg + jnp.minimum(iota, 1)])
        lo = jnp.clip(pp[0] - base2, 0, w2)
        hi = jnp.clip(pp[1] - base2, lo, w2)
        scale = plsc.load_gather(inv_v, [jnp.broadcast_to(seg, (_L,))])

        @pl.when(lo < hi)
        def _(lo=lo, hi=hi, scale=scale):
            def edge(j):
                v = chunk_v[pl.ds(off + j * _L, _L)]
                idx = j * _L + iota
                m = (idx >= lo) & (idx < hi)
                cur = out_v[pl.ds(j * _L, _L)]
                out_v[pl.ds(j * _L, _L)] = jnp.where(m, v * scale, cur)

            edge(lo // _L)
            edge((hi - 1) // _L)

            a = (lo + _L - 1) // _L
            b = jnp.maximum(a, hi // _L)

            def body2(j):
                out_v[pl.ds(j * _L, _L)] = (
                    chunk_v[pl.ds(off + j * _L, _L)] * scale)

            plsc.parallel_loop(a, b, 1, unroll=4)(body2)

        return carry

    lax.fori_loop(sf2, sl2 + 1, seg_body2, 0)

    @pl.when(jnp.logical_not(last2))
    def _():
        pltpu.sync_copy(out_v, out_hbm.at[pl.ds(base2, _C2)])

    @pl.when(last2)
    def _():
        pltpu.sync_copy(out_v.at[pl.ds(0, 800)],
                        out_hbm.at[pl.ds(_N - 800, 800)])


def kernel(node_deg, sample_pos):
    return _filtration_kernel(node_deg.astype(jnp.float32),
                              sample_pos.astype(jnp.int32))
